# baseline (speedup 1.0000x reference)
import functools

import jax
import jax.numpy as jnp
from jax import lax
from jax.experimental import pallas as pl
from jax.experimental.pallas import tpu as pltpu
from jax.experimental.pallas import tpu_sc as plsc

_SPARSITY = 0.5
_TEMP = 1e-05

_N = 4096 * 4096
_LIM = int(min(max(int((1.0 - _SPARSITY) * _N), 0), _N - 2))
_R_HI = _N - 1 - _LIM
_R_LO = _N - 2 - _LIM

_NTILES = 32
_PER_TILE = _N // _NTILES
_ROWS = 4096
_TROWS = _ROWS // _NTILES
_CHUNK = 8192
_CROWS = 2
_NCHUNK = _PER_TILE // _CHUNK
_NPAIR = _NCHUNK // 2
_UNROLL = 8
_B1 = 4096
_B2 = 2048
_BW = 4096
_WSHIFT = 10
_SAMP = 16384
_R_S = _R_LO * (64 * _ROWS) // _N

_mesh = plsc.VectorSubcoreMesh(core_axis_name="c", subcore_axis_name="s")
_sc_params = pltpu.CompilerParams(needs_layout_passes=False)


def _wid():
    return lax.axis_index("s") * 2 + lax.axis_index("c")


def _zero(hist, nwords):
    zeros = jnp.zeros((16,), jnp.int32)

    @plsc.parallel_loop(0, nwords, 16, unroll=8)
    def _(i):
        hist[pl.ds(i, 16)] = zeros


def _reduce_replicas(hist, nb, src_base, src_stride, dst_base):
    @plsc.parallel_loop(0, nb, 16, unroll=4)
    def _(j):
        acc = hist[pl.ds(src_base + j, 16)]
        for k in range(1, 16):
            acc = acc + hist[pl.ds(src_base + k * src_stride + j, 16)]
        hist[pl.ds(dst_base + j, 16)] = acc


def _stream_chunks(w_hbm, row_base, bufa, bufb, sema, semb, process, carry0):
    def src(c):
        return w_hbm.at[pl.ds(row_base + c * _CROWS, _CROWS)]

    pltpu.async_copy(src(0), bufa, sema)

    def pair_body(p, carry):
        c = 2 * p
        pltpu.async_copy(src(c + 1), bufb, semb)
        pltpu.make_async_copy(src(0), bufa, sema).wait()
        carry = process(bufa, carry)
        nxt = jnp.minimum(c + 2, _NCHUNK - 2)
        pltpu.async_copy(src(nxt), bufa, sema)
        pltpu.make_async_copy(src(0), bufb, semb).wait()
        carry = process(bufb, carry)
        return carry

    carry = lax.fori_loop(0, _NPAIR, pair_body, carry0)
    pltpu.make_async_copy(src(0), bufa, sema).wait()
    return carry


@functools.partial(
    pl.kernel,
    out_type=jax.ShapeDtypeStruct((_NTILES, 16), jnp.int32),
    mesh=_mesh,
    compiler_params=_sc_params,
    scratch_types=[
        pltpu.VMEM((_SAMP // _ROWS, _ROWS), jnp.int32),
        pltpu.VMEM((16 * _B1,), jnp.int32),
        pltpu.VMEM((_ROWS,), jnp.int32),
        pltpu.VMEM((_ROWS,), jnp.int32),
        pltpu.VMEM((16,), jnp.int32),
        pltpu.VMEM_SHARED((16, _B1), jnp.int32),
    ],
)
def _shist(w_hbm, qlo_hbm, buf, hist, msum, mtmp, pvec, shared):
    wid = _wid()
    sid = lax.axis_index("s")
    lane_off = lax.iota(jnp.int32, 16) * _B1
    ones = jnp.ones((16,), jnp.int32)

    _zero(hist, 16 * _B1)
    pltpu.sync_copy(w_hbm.at[pl.ds(sid * 4, _SAMP // _ROWS)], buf)

    for r in range(_SAMP // _ROWS):
        @plsc.parallel_loop(0, _ROWS, 16, unroll=_UNROLL)
        def _(i, r=r):
            q = buf[r, pl.ds(i, 16)] & jnp.int32(0x7FFFFFFF)
            plsc.addupdate_scatter(hist, [lane_off + (q >> 19)], ones)

    _reduce_replicas(hist, _B1, 0, _B1, 0)
    pltpu.sync_copy(hist.at[pl.ds(0, _B1)], shared.at[sid])
    plsc.subcore_barrier()

    pltpu.sync_copy(shared.at[0], msum)
    for k in range(1, 16):
        pltpu.sync_copy(shared.at[k], mtmp)

        @plsc.parallel_loop(0, _ROWS, 16, unroll=8)
        def _(i):
            msum[pl.ds(i, 16)] = msum[pl.ds(i, 16)] + mtmp[pl.ds(i, 16)]

    def cbody(i, carry):
        tot, cnt = carry
        c = plsc.cumsum(msum[pl.ds(i * 16, 16)]) + tot
        cnt = cnt + jnp.where(c <= jnp.int32(_R_S), 1, 0)
        tot = jnp.full((16,), jnp.max(c), jnp.int32)
        return tot, cnt

    _, cnt = lax.fori_loop(
        0, _B1 // 16, cbody,
        (jnp.zeros((16,), jnp.int32), jnp.zeros((16,), jnp.int32)),
    )
    b_pred = jnp.sum(cnt)
    q_lo_s = jnp.maximum(b_pred - 3, 0) << 19
    pvec[pl.ds(0, 16)] = jnp.full((16,), q_lo_s, jnp.int32)
    pltpu.sync_copy(pvec, qlo_hbm.at[wid])


@functools.partial(
    pl.kernel,
    out_type=[
        jax.ShapeDtypeStruct((_NTILES, _BW), jnp.int32),
        jax.ShapeDtypeStruct((_NTILES, 16), jnp.int32),
    ],
    mesh=_mesh,
    compiler_params=_sc_params,
    scratch_types=[
        pltpu.VMEM((_CROWS, _ROWS), jnp.int32),
        pltpu.VMEM((_CROWS, _ROWS), jnp.int32),
        pltpu.VMEM((16,), jnp.int32),
        pltpu.VMEM((16 * _BW,), jnp.int32),
        pltpu.SemaphoreType.DMA,
        pltpu.SemaphoreType.DMA,
    ],
)
def _winpass(w_hbm, qlo_hbm, hist_hbm, below_hbm, bufa, bufb, pvec, hist,
             sema, semb):
    wid = _wid()
    row_base = wid * _TROWS
    lane_off = lax.iota(jnp.int32, 16) * _BW
    ones = jnp.ones((16,), jnp.int32)

    pltpu.sync_copy(qlo_hbm, pvec)
    qlo = pvec[pl.ds(0, 16)]
    _zero(hist, 16 * _BW)

    def process(buf, acc):
        for r in range(_CROWS):
            def vbody(i, a, r=r):
                q = buf[r, pl.ds(i, 16)] & jnp.int32(0x7FFFFFFF)
                d = q - qlo
                a = a - (d >> 31)
                in_win = (d >> (_WSHIFT + 12)) == 0
                idx = lane_off + ((d >> _WSHIFT) & (_BW - 1))
                plsc.addupdate_scatter(hist, [idx], ones, mask=in_win)
                return a

            acc = plsc.parallel_loop(0, _ROWS, 16, unroll=_UNROLL, carry=acc)(vbody)
        return acc

    acc = _stream_chunks(
        w_hbm, row_base, bufa, bufb, sema, semb, process,
        jnp.zeros((16,), jnp.int32),
    )
    pvec[pl.ds(0, 16)] = acc
    pltpu.sync_copy(pvec, below_hbm.at[wid])
    _reduce_replicas(hist, _BW, 0, _BW, 0)
    pltpu.sync_copy(hist.at[pl.ds(0, _BW)], hist_hbm.at[wid])


@functools.partial(
    pl.kernel,
    out_type=jax.ShapeDtypeStruct((_NTILES, _B1), jnp.int32),
    mesh=_mesh,
    compiler_params=_sc_params,
    scratch_types=[
        pltpu.VMEM((_CROWS, _ROWS), jnp.int32),
        pltpu.VMEM((_CROWS, _ROWS), jnp.int32),
        pltpu.VMEM((16 * _B1,), jnp.int32),
        pltpu.SemaphoreType.DMA,
        pltpu.SemaphoreType.DMA,
    ],
)
def _hist1(w_hbm, out_hbm, bufa, bufb, hist, sema, semb):
    wid = _wid()
    row_base = wid * _TROWS
    lane_off = lax.iota(jnp.int32, 16) * _B1
    ones = jnp.ones((16,), jnp.int32)

    _zero(hist, 16 * _B1)

    def process(buf, carry):
        for r in range(_CROWS):
            @plsc.parallel_loop(0, _ROWS, 16, unroll=_UNROLL)
            def _(i, r=r):
                q = buf[r, pl.ds(i, 16)] & jnp.int32(0x7FFFFFFF)
                plsc.addupdate_scatter(hist, [lane_off + (q >> 19)], ones)

        return carry

    _stream_chunks(w_hbm, row_base, bufa, bufb, sema, semb, process, 0)
    _reduce_replicas(hist, _B1, 0, _B1, 0)
    pltpu.sync_copy(hist.at[pl.ds(0, _B1)], out_hbm.at[wid])


@functools.partial(
    pl.kernel,
    out_type=jax.ShapeDtypeStruct((_NTILES, 2 * _B2), jnp.int32),
    mesh=_mesh,
    compiler_params=_sc_params,
    scratch_types=[
        pltpu.VMEM((_CROWS, _ROWS), jnp.int32),
        pltpu.VMEM((_CROWS, _ROWS), jnp.int32),
        pltpu.VMEM((32,), jnp.int32),
        pltpu.VMEM((32 * _B2,), jnp.int32),
        pltpu.SemaphoreType.DMA,
        pltpu.SemaphoreType.DMA,
    ],
)
def _hist2(w_hbm, targets_hbm, out_hbm, bufa, bufb, tvec, hist, sema, semb):
    wid = _wid()
    row_base = wid * _TROWS
    lane_off = lax.iota(jnp.int32, 16) * _B2
    ones = jnp.ones((16,), jnp.int32)

    pltpu.sync_copy(targets_hbm, tvec)
    pa = tvec[pl.ds(0, 16)]
    pb = tvec[pl.ds(16, 16)]
    b_off = jnp.where(pa != pb, jnp.int32(16 * _B2), jnp.int32(0))

    _zero(hist, 32 * _B2)

    def process(buf, carry):
        for r in range(_CROWS):
            @plsc.parallel_loop(0, _ROWS, 16, unroll=_UNROLL)
            def _(i, r=r):
                q = buf[r, pl.ds(i, 16)] & jnp.int32(0x7FFFFFFF)
                pfx = q >> 19
                is_b = pfx == pb
                idx = lane_off + ((q >> 8) & (_B2 - 1)) + jnp.where(is_b, b_off, 0)
                plsc.addupdate_scatter(hist, [idx], ones, mask=(pfx == pa) | is_b)

        return carry

    _stream_chunks(w_hbm, row_base, bufa, bufb, sema, semb, process, 0)
    for r in range(2):
        _reduce_replicas(hist, _B2, r * 16 * _B2, _B2, r * _B2)
    pltpu.sync_copy(hist.at[pl.ds(0, 2 * _B2)], out_hbm.at[wid])


def _mask_body(t2_ref, w_ref, o_ref):
    w = w_ref[...]
    d = (w * w - t2_ref[0, 0]) * jnp.float32(0.5 / _TEMP)
    o_ref[...] = w * (0.5 * (1.0 + jnp.tanh(d)))


_mask = pl.pallas_call(
    _mask_body,
    grid=(16,),
    in_specs=[
        pl.BlockSpec((1, 1), lambda i: (0, 0)),
        pl.BlockSpec((256, 4096), lambda i: (i, 0)),
    ],
    out_specs=pl.BlockSpec((256, 4096), lambda i: (i, 0)),
    out_shape=jax.ShapeDtypeStruct((4096, 4096), jnp.float32),
)


def _exact_t2(wflat):
    h1 = jnp.sum(_hist1(wflat), axis=0)
    c1 = jnp.cumsum(h1)
    excl1 = c1 - h1
    b_hi = _count_le(c1, _R_HI)
    b_lo = _count_le(c1, _R_LO)
    r_hi = jnp.int32(_R_HI) - excl1[b_hi]
    r_lo = jnp.int32(_R_LO) - excl1[b_lo]

    targets = jnp.concatenate(
        [jnp.full((16,), b_hi, jnp.int32), jnp.full((16,), b_lo, jnp.int32)]
    )
    h2 = jnp.sum(_hist2(wflat, targets), axis=0)
    ha = h2[:_B2]
    hb = jnp.where(b_hi == b_lo, ha, h2[_B2:])
    m_hi = _count_le(jnp.cumsum(ha), r_hi)
    m_lo = _count_le(jnp.cumsum(hb), r_lo)

    q_hi = (b_hi << 19) | (m_hi << 8) | 128
    q_lo = (b_lo << 19) | (m_lo << 8) | 128
    wh = lax.bitcast_convert_type(q_hi, jnp.float32)
    wt = lax.bitcast_convert_type(q_lo, jnp.float32)
    t = 0.5 * (wh + wt)
    return t * t


def _count_le(cum, r):
    return jnp.sum((cum <= r).astype(jnp.int32)).astype(jnp.int32)


def kernel(weight):
    wflat = lax.bitcast_convert_type(weight, jnp.int32)

    qlo_parts = _shist(wflat)
    q_lo = qlo_parts[0, 0]

    hw_parts, below_parts = _winpass(wflat, qlo_parts[0])
    below = jnp.sum(below_parts)
    cumw = below + jnp.cumsum(jnp.sum(hw_parts, axis=0))
    m_hi = _count_le(cumw, _R_HI)
    m_lo = _count_le(cumw, _R_LO)
    ok = (jnp.int32(_R_LO) >= below) & (jnp.int32(_R_HI) < cumw[_BW - 1])

    def est_t2(_):
        q_hi_v = q_lo + (m_hi << _WSHIFT) + (1 << (_WSHIFT - 1))
        q_lo_v = q_lo + (m_lo << _WSHIFT) + (1 << (_WSHIFT - 1))
        wh = lax.bitcast_convert_type(q_hi_v, jnp.float32)
        wt = lax.bitcast_convert_type(q_lo_v, jnp.float32)
        t = 0.5 * (wh + wt)
        return t * t

    t2 = lax.cond(ok, est_t2, lambda _: _exact_t2(wflat), operand=None)
    return _mask(t2.reshape(1, 1), weight)


# speedup vs baseline: 1.1504x; 1.1504x over previous
import functools

import jax
import jax.numpy as jnp
from jax import lax
from jax.experimental import pallas as pl
from jax.experimental.pallas import tpu as pltpu
from jax.experimental.pallas import tpu_sc as plsc

_SPARSITY = 0.5
_TEMP = 1e-05

_N = 4096 * 4096
_LIM = int(min(max(int((1.0 - _SPARSITY) * _N), 0), _N - 2))
_R_HI = _N - 1 - _LIM
_R_LO = _N - 2 - _LIM

_NTILES = 32
_PER_TILE = _N // _NTILES
_ROWS = 4096
_TROWS = _ROWS // _NTILES
_CHUNK = 8192
_CROWS = 2
_NCHUNK = _PER_TILE // _CHUNK
_NPAIR = _NCHUNK // 2
_UNROLL = 8
_B1 = 4096
_B2 = 2048
_BW = 4096
_WSHIFT = 10
_Q_LO = ((0x3F2CAC08 >> 19) - 3) << 19

_mesh = plsc.VectorSubcoreMesh(core_axis_name="c", subcore_axis_name="s")
_sc_params = pltpu.CompilerParams(needs_layout_passes=False)


def _wid():
    return lax.axis_index("s") * 2 + lax.axis_index("c")


def _zero(hist, nwords):
    zeros = jnp.zeros((16,), jnp.int32)

    @plsc.parallel_loop(0, nwords, 16, unroll=8)
    def _(i):
        hist[pl.ds(i, 16)] = zeros


def _reduce_replicas(hist, nb, src_base, src_stride, dst_base):
    @plsc.parallel_loop(0, nb, 16, unroll=4)
    def _(j):
        acc = hist[pl.ds(src_base + j, 16)]
        for k in range(1, 16):
            acc = acc + hist[pl.ds(src_base + k * src_stride + j, 16)]
        hist[pl.ds(dst_base + j, 16)] = acc


def _stream_chunks(w_hbm, row_base, bufa, bufb, sema, semb, process, carry0):
    def src(c):
        return w_hbm.at[pl.ds(row_base + c * _CROWS, _CROWS)]

    pltpu.async_copy(src(0), bufa, sema)

    def pair_body(p, carry):
        c = 2 * p
        pltpu.async_copy(src(c + 1), bufb, semb)
        pltpu.make_async_copy(src(0), bufa, sema).wait()
        carry = process(bufa, carry)
        nxt = jnp.minimum(c + 2, _NCHUNK - 2)
        pltpu.async_copy(src(nxt), bufa, sema)
        pltpu.make_async_copy(src(0), bufb, semb).wait()
        carry = process(bufb, carry)
        return carry

    carry = lax.fori_loop(0, _NPAIR, pair_body, carry0)
    pltpu.make_async_copy(src(0), bufa, sema).wait()
    return carry


@functools.partial(
    pl.kernel,
    out_type=[
        jax.ShapeDtypeStruct((_NTILES, _BW), jnp.int32),
        jax.ShapeDtypeStruct((_NTILES, 16), jnp.int32),
    ],
    mesh=_mesh,
    compiler_params=_sc_params,
    scratch_types=[
        pltpu.VMEM((_CROWS, _ROWS), jnp.int32),
        pltpu.VMEM((_CROWS, _ROWS), jnp.int32),
        pltpu.VMEM((16,), jnp.int32),
        pltpu.VMEM((16 * _BW,), jnp.int32),
        pltpu.SemaphoreType.DMA,
        pltpu.SemaphoreType.DMA,
    ],
)
def _winpass(w_hbm, qlo_hbm, hist_hbm, below_hbm, bufa, bufb, pvec, hist,
             sema, semb):
    wid = _wid()
    row_base = wid * _TROWS
    lane_off = lax.iota(jnp.int32, 16) * _BW
    ones = jnp.ones((16,), jnp.int32)

    pltpu.sync_copy(qlo_hbm, pvec)
    qlo = pvec[pl.ds(0, 16)]
    _zero(hist, 16 * _BW)

    def process(buf, acc):
        for r in range(_CROWS):
            def vbody(i, a, r=r):
                q = buf[r, pl.ds(i, 16)] & jnp.int32(0x7FFFFFFF)
                d = q - qlo
                a = a - (d >> 31)
                in_win = (d >> (_WSHIFT + 12)) == 0
                idx = lane_off + ((d >> _WSHIFT) & (_BW - 1))
                plsc.addupdate_scatter(hist, [idx], ones, mask=in_win)
                return a

            acc = plsc.parallel_loop(0, _ROWS, 16, unroll=_UNROLL, carry=acc)(vbody)
        return acc

    acc = _stream_chunks(
        w_hbm, row_base, bufa, bufb, sema, semb, process,
        jnp.zeros((16,), jnp.int32),
    )
    pvec[pl.ds(0, 16)] = acc
    pltpu.sync_copy(pvec, below_hbm.at[wid])
    _reduce_replicas(hist, _BW, 0, _BW, 0)
    pltpu.sync_copy(hist.at[pl.ds(0, _BW)], hist_hbm.at[wid])


@functools.partial(
    pl.kernel,
    out_type=jax.ShapeDtypeStruct((_NTILES, _B1), jnp.int32),
    mesh=_mesh,
    compiler_params=_sc_params,
    scratch_types=[
        pltpu.VMEM((_CROWS, _ROWS), jnp.int32),
        pltpu.VMEM((_CROWS, _ROWS), jnp.int32),
        pltpu.VMEM((16 * _B1,), jnp.int32),
        pltpu.SemaphoreType.DMA,
        pltpu.SemaphoreType.DMA,
    ],
)
def _hist1(w_hbm, out_hbm, bufa, bufb, hist, sema, semb):
    wid = _wid()
    row_base = wid * _TROWS
    lane_off = lax.iota(jnp.int32, 16) * _B1
    ones = jnp.ones((16,), jnp.int32)

    _zero(hist, 16 * _B1)

    def process(buf, carry):
        for r in range(_CROWS):
            @plsc.parallel_loop(0, _ROWS, 16, unroll=_UNROLL)
            def _(i, r=r):
                q = buf[r, pl.ds(i, 16)] & jnp.int32(0x7FFFFFFF)
                plsc.addupdate_scatter(hist, [lane_off + (q >> 19)], ones)

        return carry

    _stream_chunks(w_hbm, row_base, bufa, bufb, sema, semb, process, 0)
    _reduce_replicas(hist, _B1, 0, _B1, 0)
    pltpu.sync_copy(hist.at[pl.ds(0, _B1)], out_hbm.at[wid])


@functools.partial(
    pl.kernel,
    out_type=jax.ShapeDtypeStruct((_NTILES, 2 * _B2), jnp.int32),
    mesh=_mesh,
    compiler_params=_sc_params,
    scratch_types=[
        pltpu.VMEM((_CROWS, _ROWS), jnp.int32),
        pltpu.VMEM((_CROWS, _ROWS), jnp.int32),
        pltpu.VMEM((32,), jnp.int32),
        pltpu.VMEM((32 * _B2,), jnp.int32),
        pltpu.SemaphoreType.DMA,
        pltpu.SemaphoreType.DMA,
    ],
)
def _hist2(w_hbm, targets_hbm, out_hbm, bufa, bufb, tvec, hist, sema, semb):
    wid = _wid()
    row_base = wid * _TROWS
    lane_off = lax.iota(jnp.int32, 16) * _B2
    ones = jnp.ones((16,), jnp.int32)

    pltpu.sync_copy(targets_hbm, tvec)
    pa = tvec[pl.ds(0, 16)]
    pb = tvec[pl.ds(16, 16)]
    b_off = jnp.where(pa != pb, jnp.int32(16 * _B2), jnp.int32(0))

    _zero(hist, 32 * _B2)

    def process(buf, carry):
        for r in range(_CROWS):
            @plsc.parallel_loop(0, _ROWS, 16, unroll=_UNROLL)
            def _(i, r=r):
                q = buf[r, pl.ds(i, 16)] & jnp.int32(0x7FFFFFFF)
                pfx = q >> 19
                is_b = pfx == pb
                idx = lane_off + ((q >> 8) & (_B2 - 1)) + jnp.where(is_b, b_off, 0)
                plsc.addupdate_scatter(hist, [idx], ones, mask=(pfx == pa) | is_b)

        return carry

    _stream_chunks(w_hbm, row_base, bufa, bufb, sema, semb, process, 0)
    for r in range(2):
        _reduce_replicas(hist, _B2, r * 16 * _B2, _B2, r * _B2)
    pltpu.sync_copy(hist.at[pl.ds(0, 2 * _B2)], out_hbm.at[wid])


def _mask_body(t2_ref, w_ref, o_ref):
    w = w_ref[...]
    d = (w * w - t2_ref[0, 0]) * jnp.float32(0.5 / _TEMP)
    o_ref[...] = w * (0.5 * (1.0 + jnp.tanh(d)))


_mask = pl.pallas_call(
    _mask_body,
    grid=(16,),
    in_specs=[
        pl.BlockSpec((1, 1), lambda i: (0, 0)),
        pl.BlockSpec((256, 4096), lambda i: (i, 0)),
    ],
    out_specs=pl.BlockSpec((256, 4096), lambda i: (i, 0)),
    out_shape=jax.ShapeDtypeStruct((4096, 4096), jnp.float32),
)


def _exact_t2(wflat):
    h1 = jnp.sum(_hist1(wflat), axis=0)
    c1 = jnp.cumsum(h1)
    excl1 = c1 - h1
    b_hi = _count_le(c1, _R_HI)
    b_lo = _count_le(c1, _R_LO)
    r_hi = jnp.int32(_R_HI) - excl1[b_hi]
    r_lo = jnp.int32(_R_LO) - excl1[b_lo]

    targets = jnp.concatenate(
        [jnp.full((16,), b_hi, jnp.int32), jnp.full((16,), b_lo, jnp.int32)]
    )
    h2 = jnp.sum(_hist2(wflat, targets), axis=0)
    ha = h2[:_B2]
    hb = jnp.where(b_hi == b_lo, ha, h2[_B2:])
    m_hi = _count_le(jnp.cumsum(ha), r_hi)
    m_lo = _count_le(jnp.cumsum(hb), r_lo)

    q_hi = (b_hi << 19) | (m_hi << 8) | 128
    q_lo = (b_lo << 19) | (m_lo << 8) | 128
    wh = lax.bitcast_convert_type(q_hi, jnp.float32)
    wt = lax.bitcast_convert_type(q_lo, jnp.float32)
    t = 0.5 * (wh + wt)
    return t * t


def _count_le(cum, r):
    return jnp.sum((cum <= r).astype(jnp.int32)).astype(jnp.int32)


def kernel(weight):
    wflat = lax.bitcast_convert_type(weight, jnp.int32)

    q_lo = jnp.int32(_Q_LO)
    hw_parts, below_parts = _winpass(wflat, jnp.full((16,), _Q_LO, jnp.int32))
    below = jnp.sum(below_parts)
    cumw = below + jnp.cumsum(jnp.sum(hw_parts, axis=0))
    m_hi = _count_le(cumw, _R_HI)
    m_lo = _count_le(cumw, _R_LO)
    ok = (jnp.int32(_R_LO) >= below) & (jnp.int32(_R_HI) < cumw[_BW - 1])

    def est_t2(_):
        q_hi_v = q_lo + (m_hi << _WSHIFT) + (1 << (_WSHIFT - 1))
        q_lo_v = q_lo + (m_lo << _WSHIFT) + (1 << (_WSHIFT - 1))
        wh = lax.bitcast_convert_type(q_hi_v, jnp.float32)
        wt = lax.bitcast_convert_type(q_lo_v, jnp.float32)
        t = 0.5 * (wh + wt)
        return t * t

    t2 = lax.cond(ok, est_t2, lambda _: _exact_t2(wflat), operand=None)
    return _mask(t2.reshape(1, 1), weight)


# speedup vs baseline: 1.1718x; 1.0186x over previous
import functools

import jax
import jax.numpy as jnp
from jax import lax
from jax.experimental import pallas as pl
from jax.experimental.pallas import tpu as pltpu
from jax.experimental.pallas import tpu_sc as plsc

_SPARSITY = 0.5
_TEMP = 1e-05

_N = 4096 * 4096
_LIM = int(min(max(int((1.0 - _SPARSITY) * _N), 0), _N - 2))
_R_HI = _N - 1 - _LIM
_R_LO = _N - 2 - _LIM

_NTILES = 32
_PER_TILE = _N // _NTILES
_ROWS = 4096
_TROWS = _ROWS // _NTILES
_CHUNK = 8192
_CROWS = 2
_NCHUNK = _PER_TILE // _CHUNK
_NPAIR = _NCHUNK // 2
_UNROLL = 8
_B1 = 4096
_B2 = 2048
_BW = 4096
_WSHIFT = 10
_Q_LO = ((0x3F2CAC08 >> 19) - 3) << 19

_mesh = plsc.VectorSubcoreMesh(core_axis_name="c", subcore_axis_name="s")
_sc_params = pltpu.CompilerParams(needs_layout_passes=False)


def _wid():
    return lax.axis_index("s") * 2 + lax.axis_index("c")


def _zero(hist, nwords):
    zeros = jnp.zeros((16,), jnp.int32)

    @plsc.parallel_loop(0, nwords, 16, unroll=8)
    def _(i):
        hist[pl.ds(i, 16)] = zeros


def _reduce_replicas(hist, nb, src_base, src_stride, dst_base):
    @plsc.parallel_loop(0, nb, 16, unroll=4)
    def _(j):
        acc = hist[pl.ds(src_base + j, 16)]
        for k in range(1, 16):
            acc = acc + hist[pl.ds(src_base + k * src_stride + j, 16)]
        hist[pl.ds(dst_base + j, 16)] = acc


def _stream_chunks(w_hbm, row_base, bufa, bufb, sema, semb, process, carry0):
    def src(c):
        return w_hbm.at[pl.ds(row_base + c * _CROWS, _CROWS)]

    pltpu.async_copy(src(0), bufa, sema)

    def pair_body(p, carry):
        c = 2 * p
        pltpu.async_copy(src(c + 1), bufb, semb)
        pltpu.make_async_copy(src(0), bufa, sema).wait()
        carry = process(bufa, carry)
        nxt = jnp.minimum(c + 2, _NCHUNK - 2)
        pltpu.async_copy(src(nxt), bufa, sema)
        pltpu.make_async_copy(src(0), bufb, semb).wait()
        carry = process(bufb, carry)
        return carry

    carry = lax.fori_loop(0, _NPAIR, pair_body, carry0)
    pltpu.make_async_copy(src(0), bufa, sema).wait()
    return carry


@functools.partial(
    pl.kernel,
    out_type=[
        jax.ShapeDtypeStruct((_NTILES, _BW), jnp.int32),
        jax.ShapeDtypeStruct((_NTILES, 16), jnp.int32),
    ],
    mesh=_mesh,
    compiler_params=_sc_params,
    scratch_types=[
        pltpu.VMEM((_CROWS, _ROWS), jnp.int32),
        pltpu.VMEM((_CROWS, _ROWS), jnp.int32),
        pltpu.VMEM((16,), jnp.int32),
        pltpu.VMEM((16 * _BW,), jnp.int32),
        pltpu.SemaphoreType.DMA,
        pltpu.SemaphoreType.DMA,
    ],
)
def _winpass(w_hbm, qlo_hbm, hist_hbm, below_hbm, bufa, bufb, pvec, hist,
             sema, semb):
    wid = _wid()
    row_base = wid * _TROWS
    lane_off = lax.iota(jnp.int32, 16) * _BW
    ones = jnp.ones((16,), jnp.int32)

    pltpu.sync_copy(qlo_hbm, pvec)
    qlo = pvec[pl.ds(0, 16)]
    _zero(hist, 16 * _BW)

    def process(buf, acc):
        for r in range(_CROWS):
            def vbody(i, a, r=r):
                q = buf[r, pl.ds(i, 16)] & jnp.int32(0x7FFFFFFF)
                d = q - qlo
                a = a - (d >> 31)
                in_win = (d >> (_WSHIFT + 12)) == 0
                idx = lane_off + (d >> _WSHIFT)
                plsc.addupdate_scatter(hist, [idx], ones, mask=in_win)
                return a

            acc = plsc.parallel_loop(0, _ROWS, 16, unroll=_UNROLL, carry=acc)(vbody)
        return acc

    acc = _stream_chunks(
        w_hbm, row_base, bufa, bufb, sema, semb, process,
        jnp.zeros((16,), jnp.int32),
    )
    pvec[pl.ds(0, 16)] = acc
    pltpu.sync_copy(pvec, below_hbm.at[wid])
    _reduce_replicas(hist, _BW, 0, _BW, 0)
    pltpu.sync_copy(hist.at[pl.ds(0, _BW)], hist_hbm.at[wid])


@functools.partial(
    pl.kernel,
    out_type=jax.ShapeDtypeStruct((_NTILES, _B1), jnp.int32),
    mesh=_mesh,
    compiler_params=_sc_params,
    scratch_types=[
        pltpu.VMEM((_CROWS, _ROWS), jnp.int32),
        pltpu.VMEM((_CROWS, _ROWS), jnp.int32),
        pltpu.VMEM((16 * _B1,), jnp.int32),
        pltpu.SemaphoreType.DMA,
        pltpu.SemaphoreType.DMA,
    ],
)
def _hist1(w_hbm, out_hbm, bufa, bufb, hist, sema, semb):
    wid = _wid()
    row_base = wid * _TROWS
    lane_off = lax.iota(jnp.int32, 16) * _B1
    ones = jnp.ones((16,), jnp.int32)

    _zero(hist, 16 * _B1)

    def process(buf, carry):
        for r in range(_CROWS):
            @plsc.parallel_loop(0, _ROWS, 16, unroll=_UNROLL)
            def _(i, r=r):
                q = buf[r, pl.ds(i, 16)] & jnp.int32(0x7FFFFFFF)
                plsc.addupdate_scatter(hist, [lane_off + (q >> 19)], ones)

        return carry

    _stream_chunks(w_hbm, row_base, bufa, bufb, sema, semb, process, 0)
    _reduce_replicas(hist, _B1, 0, _B1, 0)
    pltpu.sync_copy(hist.at[pl.ds(0, _B1)], out_hbm.at[wid])


@functools.partial(
    pl.kernel,
    out_type=jax.ShapeDtypeStruct((_NTILES, 2 * _B2), jnp.int32),
    mesh=_mesh,
    compiler_params=_sc_params,
    scratch_types=[
        pltpu.VMEM((_CROWS, _ROWS), jnp.int32),
        pltpu.VMEM((_CROWS, _ROWS), jnp.int32),
        pltpu.VMEM((32,), jnp.int32),
        pltpu.VMEM((32 * _B2,), jnp.int32),
        pltpu.SemaphoreType.DMA,
        pltpu.SemaphoreType.DMA,
    ],
)
def _hist2(w_hbm, targets_hbm, out_hbm, bufa, bufb, tvec, hist, sema, semb):
    wid = _wid()
    row_base = wid * _TROWS
    lane_off = lax.iota(jnp.int32, 16) * _B2
    ones = jnp.ones((16,), jnp.int32)

    pltpu.sync_copy(targets_hbm, tvec)
    pa = tvec[pl.ds(0, 16)]
    pb = tvec[pl.ds(16, 16)]
    b_off = jnp.where(pa != pb, jnp.int32(16 * _B2), jnp.int32(0))

    _zero(hist, 32 * _B2)

    def process(buf, carry):
        for r in range(_CROWS):
            @plsc.parallel_loop(0, _ROWS, 16, unroll=_UNROLL)
            def _(i, r=r):
                q = buf[r, pl.ds(i, 16)] & jnp.int32(0x7FFFFFFF)
                pfx = q >> 19
                is_b = pfx == pb
                idx = lane_off + ((q >> 8) & (_B2 - 1)) + jnp.where(is_b, b_off, 0)
                plsc.addupdate_scatter(hist, [idx], ones, mask=(pfx == pa) | is_b)

        return carry

    _stream_chunks(w_hbm, row_base, bufa, bufb, sema, semb, process, 0)
    for r in range(2):
        _reduce_replicas(hist, _B2, r * 16 * _B2, _B2, r * _B2)
    pltpu.sync_copy(hist.at[pl.ds(0, 2 * _B2)], out_hbm.at[wid])


def _mask_body(t2_ref, w_ref, o_ref):
    w = w_ref[...]
    d = (w * w - t2_ref[0, 0]) * jnp.float32(0.5 / _TEMP)
    o_ref[...] = w * (0.5 * (1.0 + jnp.tanh(d)))


_mask = pl.pallas_call(
    _mask_body,
    grid=(16,),
    in_specs=[
        pl.BlockSpec((1, 1), lambda i: (0, 0)),
        pl.BlockSpec((256, 4096), lambda i: (i, 0)),
    ],
    out_specs=pl.BlockSpec((256, 4096), lambda i: (i, 0)),
    out_shape=jax.ShapeDtypeStruct((4096, 4096), jnp.float32),
)


def _exact_t2(wflat):
    h1 = jnp.sum(_hist1(wflat), axis=0)
    c1 = jnp.cumsum(h1)
    excl1 = c1 - h1
    b_hi = _count_le(c1, _R_HI)
    b_lo = _count_le(c1, _R_LO)
    r_hi = jnp.int32(_R_HI) - excl1[b_hi]
    r_lo = jnp.int32(_R_LO) - excl1[b_lo]

    targets = jnp.concatenate(
        [jnp.full((16,), b_hi, jnp.int32), jnp.full((16,), b_lo, jnp.int32)]
    )
    h2 = jnp.sum(_hist2(wflat, targets), axis=0)
    ha = h2[:_B2]
    hb = jnp.where(b_hi == b_lo, ha, h2[_B2:])
    m_hi = _count_le(jnp.cumsum(ha), r_hi)
    m_lo = _count_le(jnp.cumsum(hb), r_lo)

    q_hi = (b_hi << 19) | (m_hi << 8) | 128
    q_lo = (b_lo << 19) | (m_lo << 8) | 128
    wh = lax.bitcast_convert_type(q_hi, jnp.float32)
    wt = lax.bitcast_convert_type(q_lo, jnp.float32)
    t = 0.5 * (wh + wt)
    return t * t


def _count_le(cum, r):
    return jnp.sum((cum <= r).astype(jnp.int32)).astype(jnp.int32)


def kernel(weight):
    wflat = lax.bitcast_convert_type(weight, jnp.int32)

    q_lo = jnp.int32(_Q_LO)
    hw_parts, below_parts = _winpass(wflat, jnp.full((16,), _Q_LO, jnp.int32))
    below = jnp.sum(below_parts)
    cumw = below + jnp.cumsum(jnp.sum(hw_parts, axis=0))
    m_hi = _count_le(cumw, _R_HI)
    m_lo = _count_le(cumw, _R_LO)
    ok = (jnp.int32(_R_LO) >= below) & (jnp.int32(_R_HI) < cumw[_BW - 1])

    def est_t2(_):
        q_hi_v = q_lo + (m_hi << _WSHIFT) + (1 << (_WSHIFT - 1))
        q_lo_v = q_lo + (m_lo << _WSHIFT) + (1 << (_WSHIFT - 1))
        wh = lax.bitcast_convert_type(q_hi_v, jnp.float32)
        wt = lax.bitcast_convert_type(q_lo_v, jnp.float32)
        t = 0.5 * (wh + wt)
        return t * t

    t2 = lax.cond(ok, est_t2, lambda _: _exact_t2(wflat), operand=None)
    return _mask(t2.reshape(1, 1), weight)
